# Initial kernel scaffold; baseline (speedup 1.0000x reference)
import jax
import jax.numpy as jnp
from jax.experimental import pallas as pl


def kernel(x, edge_index, batch, W0, b0, g0, be0, Wg, bg, gg, bgg, Wl, bl, Wt, bt):
    raise NotImplementedError("write your pallas kernel here")


# speedup vs baseline: 15.0579x; 15.0579x over previous
import functools

import jax
import jax.numpy as jnp
from jax import lax
from jax.experimental import pallas as pl
from jax.experimental.pallas import tpu as pltpu
from jax.experimental.pallas import tpu_sc as plsc

N = 10000
E = 320000
D = 128
HP = 8
NW = 32
EPW = E // NW
CH = 80
NCH = EPW // CH
RPT = N // 16


def _head_body(x_ref, w_ref, b_ref, g_ref, be_ref, h_ref):
    y = jnp.dot(x_ref[...], w_ref[...], preferred_element_type=jnp.float32)
    y = y + b_ref[...]
    mean = jnp.mean(y, axis=0, keepdims=True)
    var = jnp.mean(y * y, axis=0, keepdims=True) - mean * mean
    scale = g_ref[...] * lax.rsqrt(var + 1e-5)
    shift = be_ref[...] - mean * scale
    h_ref[...] = jnp.maximum(y * scale + shift, 0.0)


def _head(x, w, b, g, be):
    return pl.pallas_call(
        _head_body,
        out_shape=jax.ShapeDtypeStruct((N, HP), jnp.float32),
    )(x, w, b, g, be)


def _seg_body(h_hbm, src_hbm, dst_hbm, zrow_hbm, out_hbm,
              h_sh, acc_sh, idx_s, idx_d, rows, sem):
    cid = lax.axis_index("c")
    sid = lax.axis_index("s")
    sl = pl.ds(sid * RPT, RPT)
    pltpu.sync_copy(h_hbm.at[sl], h_sh.at[sl])
    pltpu.sync_copy(zrow_hbm, acc_sh.at[sl])
    plsc.subcore_barrier()
    w = sid * 2 + cid
    pltpu.sync_copy(src_hbm.at[w], idx_s)
    pltpu.sync_copy(dst_hbm.at[w], idx_d)

    def step(j, carry):
        pltpu.async_copy(h_sh.at[idx_s.at[j]], rows, sem).wait()
        pltpu.sync_copy(rows, acc_sh.at[idx_d.at[j]], add=True)
        return carry

    lax.fori_loop(0, NCH, step, 0)
    plsc.subcore_barrier()
    pltpu.sync_copy(acc_sh.at[sl], out_hbm.at[cid, sl])


def _segment_sum(h, src_r, dst_r, zrow):
    mesh = plsc.VectorSubcoreMesh(core_axis_name="c", subcore_axis_name="s")
    f = functools.partial(
        pl.kernel,
        out_type=jax.ShapeDtypeStruct((2, N, HP), jnp.float32),
        mesh=mesh,
        scratch_types=[
            pltpu.VMEM_SHARED((N, HP), jnp.float32),
            pltpu.VMEM_SHARED((N, HP), jnp.float32),
            pltpu.VMEM((NCH, CH), jnp.int32),
            pltpu.VMEM((NCH, CH), jnp.int32),
            pltpu.VMEM((CH, HP), jnp.float32),
            pltpu.SemaphoreType.DMA,
        ],
        compiler_params=pltpu.CompilerParams(use_tc_tiling_on_sc=False),
    )(_seg_body)
    return f(h, src_r, dst_r, zrow)


def _tail_body(h_ref, p_ref, wg_ref, bg_ref, gg_ref, bgg_ref,
               wl_ref, bl_ref, wt_ref, bt_ref, feat_ref, out_ref):
    m = h_ref[...] + p_ref[0] + p_ref[1]
    y = jnp.dot(m, wg_ref[...], preferred_element_type=jnp.float32)
    y = y + bg_ref[...]
    mean = jnp.mean(y, axis=0, keepdims=True)
    var = jnp.mean(y * y, axis=0, keepdims=True) - mean * mean
    scale = gg_ref[...] * lax.rsqrt(var + 1e-5)
    shift = bgg_ref[...] - mean * scale
    h2 = jnp.maximum(y * scale + shift, 0.0)
    feat = jnp.dot(h2, wl_ref[...], preferred_element_type=jnp.float32)
    feat = feat + bl_ref[...]
    feat_ref[...] = feat
    out_ref[...] = (
        jnp.dot(feat, wt_ref[...], preferred_element_type=jnp.float32)
        + bt_ref[...])


def _tail(h, part, wg, bg, gg, bgg, wl, bl, wt, bt):
    return pl.pallas_call(
        _tail_body,
        out_shape=(jax.ShapeDtypeStruct((N, HP), jnp.float32),
                   jax.ShapeDtypeStruct((N, HP), jnp.float32)),
    )(h, part, wg, bg, gg, bgg, wl, bl, wt, bt)


def kernel(x, edge_index, batch, W0, b0, g0, be0, Wg, bg, gg, bgg,
           Wl, bl, Wt, bt):
    H = W0.shape[1]
    T = Wt.shape[1]
    W08 = jnp.zeros((D, HP), jnp.float32).at[:, :H].set(W0)
    b08 = jnp.zeros((1, HP), jnp.float32).at[:, :H].set(b0)
    g08 = jnp.zeros((1, HP), jnp.float32).at[:, :H].set(g0)
    be08 = jnp.zeros((1, HP), jnp.float32).at[:, :H].set(be0)
    Wg8 = jnp.zeros((HP, HP), jnp.float32).at[:H, :H].set(Wg)
    bg8 = jnp.zeros((1, HP), jnp.float32).at[:, :H].set(bg)
    gg8 = jnp.zeros((1, HP), jnp.float32).at[:, :H].set(gg)
    bgg8 = jnp.zeros((1, HP), jnp.float32).at[:, :H].set(bgg)
    Wl8 = jnp.zeros((HP, HP), jnp.float32).at[:H, :H].set(Wl)
    bl8 = jnp.zeros((1, HP), jnp.float32).at[:, :H].set(bl)
    Wt8 = jnp.zeros((HP, HP), jnp.float32).at[:H, :T].set(Wt)
    bt8 = jnp.zeros((1, HP), jnp.float32).at[:, :T].set(bt)

    h = _head(x, W08, b08, g08, be08)

    src_r = edge_index[0].reshape(NW, NCH, CH)
    dst_r = edge_index[1].reshape(NW, NCH, CH)
    zrow = jnp.zeros((RPT, HP), jnp.float32)
    part = _segment_sum(h, src_r, dst_r, zrow)

    feat8, out8 = _tail(h, part, Wg8, bg8, gg8, bgg8, Wl8, bl8, Wt8, bt8)
    return (out8[:, :T], feat8[:, :H])


# speedup vs baseline: 18.9799x; 1.2605x over previous
import functools

import jax
import jax.numpy as jnp
from jax import lax
from jax.experimental import pallas as pl
from jax.experimental.pallas import tpu as pltpu
from jax.experimental.pallas import tpu_sc as plsc

N = 10000
E = 320000
D = 128
HP = 8
NW = 32
EPW = E // NW
CH = 80
NCH = EPW // CH
RPT = N // 16


def _head_body(x_ref, w_ref, b_ref, g_ref, be_ref, h_ref):
    w = jnp.pad(w_ref[...], ((0, 0), (0, HP - w_ref.shape[1])))
    b = jnp.pad(b_ref[...], ((0, 0), (0, HP - b_ref.shape[1])))
    g = jnp.pad(g_ref[...], ((0, 0), (0, HP - g_ref.shape[1])))
    be = jnp.pad(be_ref[...], ((0, 0), (0, HP - be_ref.shape[1])))
    y = jnp.dot(x_ref[...], w, preferred_element_type=jnp.float32) + b
    mean = jnp.mean(y, axis=0, keepdims=True)
    var = jnp.mean(y * y, axis=0, keepdims=True) - mean * mean
    scale = g * lax.rsqrt(var + 1e-5)
    shift = be - mean * scale
    h_ref[...] = jnp.maximum(y * scale + shift, 0.0)


def _head(x, w, b, g, be):
    return pl.pallas_call(
        _head_body,
        out_shape=jax.ShapeDtypeStruct((N, HP), jnp.float32),
    )(x, w, b, g, be)


U = 5
NI = NCH // U


def _seg_body(h_hbm, src_hbm, dst_hbm, zrow_hbm, out_hbm,
              h_sh, acc_sh, idx_s, idx_d, rows, stsem, gsem, ssem):
    cid = lax.axis_index("c")
    sid = lax.axis_index("s")
    sl = pl.ds(sid * RPT, RPT)
    w = sid * 2 + cid
    c1 = pltpu.async_copy(h_hbm.at[sl], h_sh.at[sl], stsem)
    c2 = pltpu.async_copy(zrow_hbm, acc_sh.at[sl], stsem)
    c3 = pltpu.async_copy(src_hbm.at[w], idx_s, gsem)
    c4 = pltpu.async_copy(dst_hbm.at[w], idx_d, gsem)
    c1.wait(); c2.wait()
    plsc.subcore_barrier()
    c3.wait(); c4.wait()

    def step(i, carry):
        base = i * U
        gcps = [pltpu.async_copy(h_sh.at[idx_s.at[base + u]],
                                 rows.at[pl.ds(u * CH, CH)], gsem)
                for u in range(U)]
        for cp in gcps:
            cp.wait()
        scps = [pltpu.async_copy(rows.at[pl.ds(u * CH, CH)],
                                 acc_sh.at[idx_d.at[base + u]], ssem,
                                 add=True)
                for u in range(U)]
        for cp in scps:
            cp.wait()
        return carry

    lax.fori_loop(0, NI, step, 0)
    plsc.subcore_barrier()
    pltpu.sync_copy(acc_sh.at[sl], out_hbm.at[cid, sl])


def _segment_sum(h, src_r, dst_r, zrow):
    mesh = plsc.VectorSubcoreMesh(core_axis_name="c", subcore_axis_name="s")
    f = functools.partial(
        pl.kernel,
        out_type=jax.ShapeDtypeStruct((2, N, HP), jnp.float32),
        mesh=mesh,
        scratch_types=[
            pltpu.VMEM_SHARED((N, HP), jnp.float32),
            pltpu.VMEM_SHARED((N, HP), jnp.float32),
            pltpu.VMEM((NCH, CH), jnp.int32),
            pltpu.VMEM((NCH, CH), jnp.int32),
            pltpu.VMEM((U * CH, HP), jnp.float32),
            pltpu.SemaphoreType.DMA,
            pltpu.SemaphoreType.DMA,
            pltpu.SemaphoreType.DMA,
        ],
        compiler_params=pltpu.CompilerParams(use_tc_tiling_on_sc=False),
    )(_seg_body)
    return f(h, src_r, dst_r, zrow)


def _tail_body(h_ref, p_ref, wg_ref, bg_ref, gg_ref, bgg_ref,
               wl_ref, bl_ref, wt_ref, bt_ref, out_ref, feat_ref):
    hpad = HP - wg_ref.shape[0]
    wg = jnp.pad(wg_ref[...], ((0, hpad), (0, hpad)))
    bg = jnp.pad(bg_ref[...], ((0, 0), (0, hpad)))
    gg = jnp.pad(gg_ref[...], ((0, 0), (0, hpad)))
    bgg = jnp.pad(bgg_ref[...], ((0, 0), (0, hpad)))
    wl = jnp.pad(wl_ref[...], ((0, hpad), (0, hpad)))
    bl = jnp.pad(bl_ref[...], ((0, 0), (0, hpad)))
    wt = jnp.pad(wt_ref[...], ((0, hpad), (0, 0)))
    m = h_ref[...] + p_ref[0] + p_ref[1]
    y = jnp.dot(m, wg, preferred_element_type=jnp.float32) + bg
    mean = jnp.mean(y, axis=0, keepdims=True)
    var = jnp.mean(y * y, axis=0, keepdims=True) - mean * mean
    scale = gg * lax.rsqrt(var + 1e-5)
    shift = bgg - mean * scale
    h2 = jnp.maximum(y * scale + shift, 0.0)
    feat = jnp.dot(h2, wl, preferred_element_type=jnp.float32) + bl
    feat_ref[...] = feat[:, :feat_ref.shape[1]]
    out_ref[...] = (
        jnp.dot(feat, wt, preferred_element_type=jnp.float32)
        + bt_ref[...])


def _tail(h, part, wg, bg, gg, bgg, wl, bl, wt, bt, hdim, tdim):
    return pl.pallas_call(
        _tail_body,
        out_shape=(jax.ShapeDtypeStruct((N, tdim), jnp.float32),
                   jax.ShapeDtypeStruct((N, hdim), jnp.float32)),
    )(h, part, wg, bg, gg, bgg, wl, bl, wt, bt)


def kernel(x, edge_index, batch, W0, b0, g0, be0, Wg, bg, gg, bgg,
           Wl, bl, Wt, bt):
    H = W0.shape[1]
    T = Wt.shape[1]
    h = _head(x, W0, b0.reshape(1, H), g0.reshape(1, H), be0.reshape(1, H))

    src_r = edge_index[0].reshape(NW, NCH, CH)
    dst_r = edge_index[1].reshape(NW, NCH, CH)
    zrow = jnp.zeros((RPT, HP), jnp.float32)
    part = _segment_sum(h, src_r, dst_r, zrow)

    out, feat = _tail(h, part, Wg, bg.reshape(1, H), gg.reshape(1, H),
                      bgg.reshape(1, H), Wl, bl.reshape(1, H), Wt,
                      bt.reshape(1, T), H, T)
    return (out, feat)


# speedup vs baseline: 19.1652x; 1.0098x over previous
import functools

import jax
import jax.numpy as jnp
from jax import lax
from jax.experimental import pallas as pl
from jax.experimental.pallas import tpu as pltpu
from jax.experimental.pallas import tpu_sc as plsc

N = 10000
E = 320000
D = 128
HP = 8
NW = 32
EPW = E // NW
CH = 80
NCH = EPW // CH
RPT = N // 16


def _head_body(x_ref, w_ref, b_ref, g_ref, be_ref, h_ref):
    w = jnp.pad(w_ref[...], ((0, 0), (0, HP - w_ref.shape[1])))
    b = jnp.pad(b_ref[...], ((0, 0), (0, HP - b_ref.shape[1])))
    g = jnp.pad(g_ref[...], ((0, 0), (0, HP - g_ref.shape[1])))
    be = jnp.pad(be_ref[...], ((0, 0), (0, HP - be_ref.shape[1])))
    y = jnp.dot(x_ref[...], w, preferred_element_type=jnp.float32) + b
    mean = jnp.mean(y, axis=0, keepdims=True)
    var = jnp.mean(y * y, axis=0, keepdims=True) - mean * mean
    scale = g * lax.rsqrt(var + 1e-5)
    shift = be - mean * scale
    h_ref[...] = jnp.maximum(y * scale + shift, 0.0)


def _head(x, w, b, g, be):
    return pl.pallas_call(
        _head_body,
        out_shape=jax.ShapeDtypeStruct((N, HP), jnp.float32),
    )(x, w, b, g, be)


U = 5
NI = NCH // U


def _seg_body(h_hbm, src_hbm, dst_hbm, zrow_hbm, out_hbm,
              h_sh, acc_sh, idx_s, idx_d, rows, stsem, gsem, ssem):
    cid = lax.axis_index("c")
    sid = lax.axis_index("s")
    sl = pl.ds(sid * RPT, RPT)
    w = sid * 2 + cid
    c1 = pltpu.async_copy(h_hbm.at[sl], h_sh.at[sl], stsem)

    @pl.when(cid == 0)
    def _():
        pltpu.async_copy(h_hbm.at[sl], acc_sh.at[sl], stsem)

    @pl.when(cid == 1)
    def _():
        pltpu.async_copy(zrow_hbm, acc_sh.at[sl], stsem)

    c3 = pltpu.async_copy(src_hbm.at[w], idx_s, gsem)
    c4 = pltpu.async_copy(dst_hbm.at[w], idx_d, gsem)
    c1.wait()
    pltpu.make_async_copy(zrow_hbm, acc_sh.at[sl], stsem).wait()
    plsc.subcore_barrier()
    c3.wait(); c4.wait()

    def step(i, carry):
        base = i * U
        gcps = [pltpu.async_copy(h_sh.at[idx_s.at[base + u]],
                                 rows.at[pl.ds(u * CH, CH)], gsem)
                for u in range(U)]
        for cp in gcps:
            cp.wait()
        scps = [pltpu.async_copy(rows.at[pl.ds(u * CH, CH)],
                                 acc_sh.at[idx_d.at[base + u]], ssem,
                                 add=True)
                for u in range(U)]
        for cp in scps:
            cp.wait()
        return carry

    lax.fori_loop(0, NI, step, 0)
    plsc.subcore_barrier()
    pltpu.sync_copy(acc_sh.at[sl], out_hbm.at[cid, sl])


def _segment_sum(h, src_r, dst_r, zrow):
    mesh = plsc.VectorSubcoreMesh(core_axis_name="c", subcore_axis_name="s")
    f = functools.partial(
        pl.kernel,
        out_type=jax.ShapeDtypeStruct((2, N, HP), jnp.float32),
        mesh=mesh,
        scratch_types=[
            pltpu.VMEM_SHARED((N, HP), jnp.float32),
            pltpu.VMEM_SHARED((N, HP), jnp.float32),
            pltpu.VMEM((NCH, CH), jnp.int32),
            pltpu.VMEM((NCH, CH), jnp.int32),
            pltpu.VMEM((U * CH, HP), jnp.float32),
            pltpu.SemaphoreType.DMA,
            pltpu.SemaphoreType.DMA,
            pltpu.SemaphoreType.DMA,
        ],
        compiler_params=pltpu.CompilerParams(use_tc_tiling_on_sc=False),
    )(_seg_body)
    return f(h, src_r, dst_r, zrow)


def _tail_body(h_ref, p_ref, wg_ref, bg_ref, gg_ref, bgg_ref,
               wl_ref, bl_ref, wt_ref, bt_ref, out_ref, feat_ref):
    hpad = HP - wg_ref.shape[0]
    wg = jnp.pad(wg_ref[...], ((0, hpad), (0, hpad)))
    bg = jnp.pad(bg_ref[...], ((0, 0), (0, hpad)))
    gg = jnp.pad(gg_ref[...], ((0, 0), (0, hpad)))
    bgg = jnp.pad(bgg_ref[...], ((0, 0), (0, hpad)))
    wl = jnp.pad(wl_ref[...], ((0, hpad), (0, hpad)))
    bl = jnp.pad(bl_ref[...], ((0, 0), (0, hpad)))
    wt = jnp.pad(wt_ref[...], ((0, hpad), (0, 0)))
    m = p_ref[0] + p_ref[1]
    y = jnp.dot(m, wg, preferred_element_type=jnp.float32) + bg
    mean = jnp.mean(y, axis=0, keepdims=True)
    var = jnp.mean(y * y, axis=0, keepdims=True) - mean * mean
    scale = gg * lax.rsqrt(var + 1e-5)
    shift = bgg - mean * scale
    h2 = jnp.maximum(y * scale + shift, 0.0)
    feat = jnp.dot(h2, wl, preferred_element_type=jnp.float32) + bl
    feat_ref[...] = feat[:, :feat_ref.shape[1]]
    out_ref[...] = (
        jnp.dot(feat, wt, preferred_element_type=jnp.float32)
        + bt_ref[...])


def _tail(h, part, wg, bg, gg, bgg, wl, bl, wt, bt, hdim, tdim):
    return pl.pallas_call(
        _tail_body,
        out_shape=(jax.ShapeDtypeStruct((N, tdim), jnp.float32),
                   jax.ShapeDtypeStruct((N, hdim), jnp.float32)),
    )(h, part, wg, bg, gg, bgg, wl, bl, wt, bt)


def kernel(x, edge_index, batch, W0, b0, g0, be0, Wg, bg, gg, bgg,
           Wl, bl, Wt, bt):
    H = W0.shape[1]
    T = Wt.shape[1]
    h = _head(x, W0, b0.reshape(1, H), g0.reshape(1, H), be0.reshape(1, H))

    src_r = edge_index[0].reshape(NW, NCH, CH)
    dst_r = edge_index[1].reshape(NW, NCH, CH)
    zrow = jnp.zeros((RPT, HP), jnp.float32)
    part = _segment_sum(h, src_r, dst_r, zrow)

    out, feat = _tail(h, part, Wg, bg.reshape(1, H), gg.reshape(1, H),
                      bgg.reshape(1, H), Wl, bl.reshape(1, H), Wt,
                      bt.reshape(1, T), H, T)
    return (out, feat)
